# baseline (device time: 30199 ns/iter reference)
import jax
import jax.numpy as jnp
from jax import lax
from jax.experimental import pallas as pl
from jax.experimental.pallas import tpu as pltpu

N_DEV = 16
B = 2
SQ = 128
D = 512
HQ_LOC = 8
DH = 64
GQA = 4
HKV = HQ_LOC // GQA
R = B * SQ
DC = D // 2


def kernel(x, Wq, Wo, K_ext, V_ext):
    idx = lax.axis_index("i")
    K_loc = jnp.reshape(
        lax.dynamic_slice_in_dim(K_ext, idx * HKV, HKV, axis=2), (B, SQ, HKV * DH))
    V_loc = jnp.reshape(
        lax.dynamic_slice_in_dim(V_ext, idx * HKV, HKV, axis=2), (B, SQ, HKV * DH))

    def body(x_ref, wq_ref, wo_ref, k_ref, v_ref, out_ref, att_ref,
             pstA, rst0A, wmidA, rst1A, wfinA, gmidA,
             pstB, rst0B, wmidB, rst1B, wfinB, gmidB,
             own_sems,
             rs_sendA, rs_recvA, ag_sendA, ag_recvA,
             rs_sendB, rs_recvB, ag_sendB, ag_recvB):
        my = lax.axis_index("i")

        barrier_sem = pltpu.get_barrier_semaphore()
        for d in (1, 2, 3, 4, 8, 12):
            pl.semaphore_signal(
                barrier_sem, inc=1,
                device_id=(my ^ d,), device_id_type=pl.DeviceIdType.MESH,
            )

        for b in range(B):
            qb = jnp.dot(x_ref[b], wq_ref[...],
                         preferred_element_type=jnp.float32)
            for h in range(HQ_LOC):
                c = h // GQA
                kb = k_ref[b, :, c * DH:(c + 1) * DH]
                vb = v_ref[b, :, c * DH:(c + 1) * DH]
                qh = qb[:, h * DH:(h + 1) * DH]
                s = lax.dot_general(
                    qh, kb, (((1,), (1,)), ((), ())),
                    preferred_element_type=jnp.float32,
                ) * 0.125
                m = jnp.max(s, axis=-1, keepdims=True)
                p = jnp.exp(s - m)
                l = jnp.sum(p, axis=-1, keepdims=True)
                o = jnp.dot(p, vb, preferred_element_type=jnp.float32) / l
                att_ref[b, :, h * DH:(h + 1) * DH] = o
            pstA[pl.ds(b * SQ, SQ), :] = jnp.dot(
                att_ref[b], wo_ref[:, :DC],
                preferred_element_type=jnp.float32)

        pl.semaphore_wait(barrier_sem, 6)

        drains = []

        def rs4_start(w_in, rst, S, dbase, send_sems, recv_sems, base):
            blk = S // 4
            t = lax.rem(my // dbase, 4)
            wds = []
            for j in (1, 2, 3):
                partner = my ^ (j * dbase)
                ts = t ^ j
                pltpu.make_async_remote_copy(
                    src_ref=w_in.at[pl.ds(ts * blk, blk)],
                    dst_ref=rst.at[j - 1],
                    send_sem=send_sems.at[base + j - 1],
                    recv_sem=recv_sems.at[base + j - 1],
                    device_id=(partner,), device_id_type=pl.DeviceIdType.MESH,
                ).start()
                wds.append(pltpu.make_async_remote_copy(
                    src_ref=rst.at[j - 1], dst_ref=rst.at[j - 1],
                    send_sem=send_sems.at[base + j - 1],
                    recv_sem=recv_sems.at[base + j - 1],
                    device_id=(partner,), device_id_type=pl.DeviceIdType.MESH,
                ))
            return wds, t, blk

        def rs4_finish(wds, t, blk, w_in, rst, w_out):
            for wd in wds:
                wd.wait_recv()
            w_out[...] = (w_in[pl.ds(t * blk, blk), :]
                          + rst[0] + rst[1] + rst[2])
            drains.extend(wds)

        def ag4_start(cur, slicer, blk, dbase, send_sems, recv_sems, base,
                      own_sem):
            t = lax.rem(my // dbase, 4)
            own = pltpu.make_async_copy(cur, slicer(t), own_sem)
            own.start()
            wds = []
            for j in (1, 2, 3):
                partner = my ^ (j * dbase)
                pltpu.make_async_remote_copy(
                    src_ref=cur, dst_ref=slicer(t),
                    send_sem=send_sems.at[base + j - 1],
                    recv_sem=recv_sems.at[base + j - 1],
                    device_id=(partner,), device_id_type=pl.DeviceIdType.MESH,
                ).start()
                wds.append(pltpu.make_async_remote_copy(
                    src_ref=cur, dst_ref=slicer(0),
                    send_sem=send_sems.at[base + j - 1],
                    recv_sem=recv_sems.at[base + j - 1],
                    device_id=(partner,), device_id_type=pl.DeviceIdType.MESH,
                ))
            return wds, own

        def ag4_finish(wds, own):
            for wd in wds:
                wd.wait_recv()
            own.wait()
            drains.extend(wds)

        wdsA, tA, blkA = rs4_start(pstA, rst0A, R, 1, rs_sendA, rs_recvA, 0)
        for b in range(B):
            pstB[pl.ds(b * SQ, SQ), :] = jnp.dot(
                att_ref[b], wo_ref[:, DC:],
                preferred_element_type=jnp.float32)
        wdsB, tB, blkB = rs4_start(pstB, rst0B, R, 4, rs_sendB, rs_recvB, 0)
        rs4_finish(wdsA, tA, blkA, pstA, rst0A, wmidA)
        rs4_finish(wdsB, tB, blkB, pstB, rst0B, wmidB)
        wdsA, tA, blkA = rs4_start(wmidA, rst1A, R // 4, 4, rs_sendA, rs_recvA, 3)
        wdsB, tB, blkB = rs4_start(wmidB, rst1B, R // 4, 1, rs_sendB, rs_recvB, 3)
        rs4_finish(wdsA, tA, blkA, wmidA, rst1A, wfinA)
        rs4_finish(wdsB, tB, blkB, wmidB, rst1B, wfinB)

        blk1 = R // 16
        wdsA, ownA = ag4_start(
            wfinA, lambda t: gmidA.at[pl.ds(t * blk1, blk1)], blk1, 4,
            ag_sendA, ag_recvA, 0, own_sems.at[0])
        wdsB, ownB = ag4_start(
            wfinB, lambda t: gmidB.at[pl.ds(t * blk1, blk1)], blk1, 1,
            ag_sendB, ag_recvB, 0, own_sems.at[1])
        ag4_finish(wdsA, ownA)
        ag4_finish(wdsB, ownB)

        blk0 = R // 4
        def out_slicer(col0):
            return lambda t: out_ref.at[
                t // 2, pl.ds(lax.rem(t, 2) * blk0, blk0), pl.ds(col0, DC)]
        wdsA, ownA = ag4_start(
            gmidA, out_slicer(0), blk0, 1,
            ag_sendA, ag_recvA, 3, own_sems.at[2])
        wdsB, ownB = ag4_start(
            gmidB, out_slicer(DC), blk0, 4,
            ag_sendB, ag_recvB, 3, own_sems.at[3])
        ag4_finish(wdsA, ownA)
        ag4_finish(wdsB, ownB)

        for wd in drains:
            wd.wait_send()

    def stream_bufs():
        return [
            pltpu.VMEM((R, DC), jnp.float32),
            pltpu.VMEM((3, R // 4, DC), jnp.float32),
            pltpu.VMEM((R // 4, DC), jnp.float32),
            pltpu.VMEM((3, R // 16, DC), jnp.float32),
            pltpu.VMEM((R // 16, DC), jnp.float32),
            pltpu.VMEM((R // 4, DC), jnp.float32),
        ]

    return pl.pallas_call(
        body,
        out_shape=jax.ShapeDtypeStruct((B, SQ, D), jnp.float32),
        in_specs=[pl.BlockSpec(memory_space=pltpu.VMEM)] * 5,
        out_specs=pl.BlockSpec(memory_space=pltpu.VMEM),
        scratch_shapes=(
            [pltpu.VMEM((B, SQ, D), jnp.float32)]
            + stream_bufs() + stream_bufs()
            + [pltpu.SemaphoreType.DMA((4,))]
            + [pltpu.SemaphoreType.DMA((6,))] * 8
        ),
        compiler_params=pltpu.CompilerParams(collective_id=0),
    )(x, Wq, Wo, K_loc, V_loc)


# device time: 29897 ns/iter; 1.0101x vs baseline; 1.0101x over previous
import jax
import jax.numpy as jnp
from jax import lax
from jax.experimental import pallas as pl
from jax.experimental.pallas import tpu as pltpu

N_DEV = 16
B = 2
SQ = 128
D = 512
HQ_LOC = 8
DH = 64
GQA = 4
HKV = HQ_LOC // GQA
R = B * SQ
DC = D // 2


def kernel(x, Wq, Wo, K_ext, V_ext):
    idx = lax.axis_index("i")
    K_loc = jnp.reshape(
        lax.dynamic_slice_in_dim(K_ext, idx * HKV, HKV, axis=2), (B, SQ, HKV * DH))
    V_loc = jnp.reshape(
        lax.dynamic_slice_in_dim(V_ext, idx * HKV, HKV, axis=2), (B, SQ, HKV * DH))

    def body(x_ref, wq_ref, wo_ref, k_ref, v_ref, out_ref, att_ref,
             pstA, rst0A, wmidA, rst1A, wfinA, gmidA,
             pstB, rst0B, wmidB, rst1B, wfinB, gmidB,
             own_sems,
             rs_sendA, rs_recvA, ag_sendA, ag_recvA,
             rs_sendB, rs_recvB, ag_sendB, ag_recvB):
        my = lax.axis_index("i")

        barrier_sem = pltpu.get_barrier_semaphore()
        for d in (1, 2, 3, 4, 8, 12):
            pl.semaphore_signal(
                barrier_sem, inc=1,
                device_id=(my ^ d,), device_id_type=pl.DeviceIdType.MESH,
            )

        for b in range(B):
            qb = jnp.dot(x_ref[b], wq_ref[...],
                         preferred_element_type=jnp.float32)
            for h in range(HQ_LOC):
                c = h // GQA
                kb = k_ref[b, :, c * DH:(c + 1) * DH]
                vb = v_ref[b, :, c * DH:(c + 1) * DH]
                qh = qb[:, h * DH:(h + 1) * DH]
                s = lax.dot_general(
                    qh, kb, (((1,), (1,)), ((), ())),
                    preferred_element_type=jnp.float32,
                ) * 0.125
                m = jnp.max(s, axis=-1, keepdims=True)
                p = jnp.exp(s - m)
                l = jnp.sum(p, axis=-1, keepdims=True)
                o = jnp.dot(p, vb, preferred_element_type=jnp.float32) / l
                att_ref[b, :, h * DH:(h + 1) * DH] = o
            part = jnp.dot(att_ref[b], wo_ref[...],
                           preferred_element_type=jnp.float32)
            pstA[pl.ds(b * SQ, SQ), :] = part[:, :DC]
            pstB[pl.ds(b * SQ, SQ), :] = part[:, DC:]

        pl.semaphore_wait(barrier_sem, 6)

        drains = []

        def rs4_start(w_in, rst, S, dbase, send_sems, recv_sems, base):
            blk = S // 4
            t = lax.rem(my // dbase, 4)
            wds = []
            for j in (1, 2, 3):
                partner = my ^ (j * dbase)
                ts = t ^ j
                pltpu.make_async_remote_copy(
                    src_ref=w_in.at[pl.ds(ts * blk, blk)],
                    dst_ref=rst.at[j - 1],
                    send_sem=send_sems.at[base + j - 1],
                    recv_sem=recv_sems.at[base + j - 1],
                    device_id=(partner,), device_id_type=pl.DeviceIdType.MESH,
                ).start()
                wds.append(pltpu.make_async_remote_copy(
                    src_ref=rst.at[j - 1], dst_ref=rst.at[j - 1],
                    send_sem=send_sems.at[base + j - 1],
                    recv_sem=recv_sems.at[base + j - 1],
                    device_id=(partner,), device_id_type=pl.DeviceIdType.MESH,
                ))
            return wds, t, blk

        def rs4_finish(wds, t, blk, w_in, rst, w_out):
            for wd in wds:
                wd.wait_recv()
            w_out[...] = (w_in[pl.ds(t * blk, blk), :]
                          + rst[0] + rst[1] + rst[2])
            drains.extend(wds)

        def ag4_start(cur, slicer, blk, dbase, send_sems, recv_sems, base,
                      own_sem):
            t = lax.rem(my // dbase, 4)
            own = pltpu.make_async_copy(cur, slicer(t), own_sem)
            own.start()
            wds = []
            for j in (1, 2, 3):
                partner = my ^ (j * dbase)
                pltpu.make_async_remote_copy(
                    src_ref=cur, dst_ref=slicer(t),
                    send_sem=send_sems.at[base + j - 1],
                    recv_sem=recv_sems.at[base + j - 1],
                    device_id=(partner,), device_id_type=pl.DeviceIdType.MESH,
                ).start()
                wds.append(pltpu.make_async_remote_copy(
                    src_ref=cur, dst_ref=slicer(0),
                    send_sem=send_sems.at[base + j - 1],
                    recv_sem=recv_sems.at[base + j - 1],
                    device_id=(partner,), device_id_type=pl.DeviceIdType.MESH,
                ))
            return wds, own

        def ag4_finish(wds, own):
            for wd in wds:
                wd.wait_recv()
            own.wait()
            drains.extend(wds)

        wdsA, tA, blkA = rs4_start(pstA, rst0A, R, 1, rs_sendA, rs_recvA, 0)
        wdsB, tB, blkB = rs4_start(pstB, rst0B, R, 4, rs_sendB, rs_recvB, 0)
        rs4_finish(wdsA, tA, blkA, pstA, rst0A, wmidA)
        rs4_finish(wdsB, tB, blkB, pstB, rst0B, wmidB)
        wdsA, tA, blkA = rs4_start(wmidA, rst1A, R // 4, 4, rs_sendA, rs_recvA, 3)
        wdsB, tB, blkB = rs4_start(wmidB, rst1B, R // 4, 1, rs_sendB, rs_recvB, 3)
        rs4_finish(wdsA, tA, blkA, wmidA, rst1A, wfinA)
        rs4_finish(wdsB, tB, blkB, wmidB, rst1B, wfinB)

        blk1 = R // 16
        wdsA, ownA = ag4_start(
            wfinA, lambda t: gmidA.at[pl.ds(t * blk1, blk1)], blk1, 4,
            ag_sendA, ag_recvA, 0, own_sems.at[0])
        wdsB, ownB = ag4_start(
            wfinB, lambda t: gmidB.at[pl.ds(t * blk1, blk1)], blk1, 1,
            ag_sendB, ag_recvB, 0, own_sems.at[1])
        ag4_finish(wdsA, ownA)
        ag4_finish(wdsB, ownB)

        blk0 = R // 4
        def out_slicer(col0):
            return lambda t: out_ref.at[
                t // 2, pl.ds(lax.rem(t, 2) * blk0, blk0), pl.ds(col0, DC)]
        wdsA, ownA = ag4_start(
            gmidA, out_slicer(0), blk0, 1,
            ag_sendA, ag_recvA, 3, own_sems.at[2])
        wdsB, ownB = ag4_start(
            gmidB, out_slicer(DC), blk0, 4,
            ag_sendB, ag_recvB, 3, own_sems.at[3])
        ag4_finish(wdsA, ownA)
        ag4_finish(wdsB, ownB)

        for wd in drains:
            wd.wait_send()

    def stream_bufs():
        return [
            pltpu.VMEM((R, DC), jnp.float32),
            pltpu.VMEM((3, R // 4, DC), jnp.float32),
            pltpu.VMEM((R // 4, DC), jnp.float32),
            pltpu.VMEM((3, R // 16, DC), jnp.float32),
            pltpu.VMEM((R // 16, DC), jnp.float32),
            pltpu.VMEM((R // 4, DC), jnp.float32),
        ]

    return pl.pallas_call(
        body,
        out_shape=jax.ShapeDtypeStruct((B, SQ, D), jnp.float32),
        in_specs=[pl.BlockSpec(memory_space=pltpu.VMEM)] * 5,
        out_specs=pl.BlockSpec(memory_space=pltpu.VMEM),
        scratch_shapes=(
            [pltpu.VMEM((B, SQ, D), jnp.float32)]
            + stream_bufs() + stream_bufs()
            + [pltpu.SemaphoreType.DMA((4,))]
            + [pltpu.SemaphoreType.DMA((6,))] * 8
        ),
        compiler_params=pltpu.CompilerParams(collective_id=0),
    )(x, Wq, Wo, K_loc, V_loc)
